# baseline (speedup 1.0000x reference)
import functools

import jax
import jax.numpy as jnp
from jax import lax
from jax.experimental import pallas as pl
from jax.experimental.pallas import tpu as pltpu
from jax.experimental.pallas import tpu_sc as plsc

BATCH = 4096
SEQ = 50
DIM = 64
B = BATCH * SEQ
NC = 2
NS = 16
NW = NC * NS
CPW = BATCH // NW
CH = CPW
HD = DIM // 2
G = 5
NGROUPS = SEQ // G

_mesh = plsc.VectorSubcoreMesh(
    core_axis_name="c", subcore_axis_name="s", num_cores=NC, num_subcores=NS
)


@functools.partial(
    pl.kernel,
    out_type=jax.ShapeDtypeStruct((B, DIM), jnp.float32),
    mesh=_mesh,
    scratch_types=[
        pltpu.VMEM((SEQ, CH), jnp.int32),
        pltpu.VMEM((2 * G, CH, HD), jnp.float32),
        pltpu.VMEM((2 * G, CH, HD), jnp.float32),
        pltpu.SemaphoreType.DMA,
        pltpu.SemaphoreType.DMA,
    ],
    compiler_params=pltpu.CompilerParams(use_tc_tiling_on_sc=False),
)
def _emb_lookup(idx_hbm, tab_a, tab_b, out_hbm, idx_v, ra_v, rb_v, gsem, ssem):
    wid = lax.axis_index("s") * NC + lax.axis_index("c")
    col = wid * CPW
    pltpu.sync_copy(idx_hbm.at[:, pl.ds(col, CPW)], idx_v)

    for b in range(G):
        pltpu.async_copy(tab_a.at[idx_v.at[b]], ra_v.at[b], gsem)
        pltpu.async_copy(tab_b.at[idx_v.at[b]], rb_v.at[b], gsem)

    @pl.loop(0, NGROUPS)
    def _(k):
        bank = lax.rem(k, 2) * G
        nbank = G - bank

        for b in range(2 * G):
            pltpu.make_async_copy(
                tab_a.at[idx_v.at[0]], ra_v.at[0], gsem
            ).wait()

        @pl.when(k >= 1)
        def _():
            for b in range(2 * G):
                pltpu.make_async_copy(
                    ra_v.at[0],
                    out_hbm.at[pl.ds(col, CH), pl.ds(0, HD)],
                    ssem,
                ).wait()

        @pl.when(k + 1 < NGROUPS)
        def _():
            for b in range(G):
                pltpu.async_copy(
                    tab_a.at[idx_v.at[(k + 1) * G + b]],
                    ra_v.at[nbank + b],
                    gsem,
                )
                pltpu.async_copy(
                    tab_b.at[idx_v.at[(k + 1) * G + b]],
                    rb_v.at[nbank + b],
                    gsem,
                )

        for b in range(G):
            row0 = (k * G + b) * BATCH + col
            pltpu.async_copy(
                ra_v.at[bank + b],
                out_hbm.at[pl.ds(row0, CH), pl.ds(0, HD)],
                ssem,
            )
            pltpu.async_copy(
                rb_v.at[bank + b],
                out_hbm.at[pl.ds(row0, CH), pl.ds(HD, HD)],
                ssem,
            )

    for b in range(2 * G):
        pltpu.make_async_copy(
            ra_v.at[0], out_hbm.at[pl.ds(col, CH), pl.ds(0, HD)], ssem
        ).wait()


def kernel(token_ids, table):
    idx_t = token_ids.T.astype(jnp.int32)
    out = _emb_lookup(idx_t, table[:, :HD], table[:, HD:])
    return out.reshape(SEQ, BATCH, DIM).transpose(1, 0, 2)


# speedup vs baseline: 2.0237x; 2.0237x over previous
import functools

import jax
import jax.numpy as jnp
from jax import lax
from jax.experimental import pallas as pl
from jax.experimental.pallas import tpu as pltpu
from jax.experimental.pallas import tpu_sc as plsc

BATCH = 4096
SEQ = 50
DIM = 64
B = BATCH * SEQ
NC = 2
NS = 16
NW = NC * NS
CPW = BATCH // NW
CH = CPW
G = 5
NGROUPS = SEQ // G

_mesh = plsc.VectorSubcoreMesh(
    core_axis_name="c", subcore_axis_name="s", num_cores=NC, num_subcores=NS
)


@functools.partial(
    pl.kernel,
    out_type=jax.ShapeDtypeStruct((B, DIM), jnp.float32),
    mesh=_mesh,
    scratch_types=[
        pltpu.VMEM((SEQ, CH), jnp.int32),
        pltpu.VMEM((2 * G, CH, DIM), jnp.float32),
        pltpu.SemaphoreType.DMA,
        pltpu.SemaphoreType.DMA,
    ],
    compiler_params=pltpu.CompilerParams(use_tc_tiling_on_sc=False),
)
def _emb_lookup(idx_hbm, table_hbm, out_hbm, idx_v, rows_v, gsem, ssem):
    wid = lax.axis_index("s") * NC + lax.axis_index("c")
    col = wid * CPW
    pltpu.sync_copy(idx_hbm.at[:, pl.ds(col, CPW)], idx_v)

    for b in range(G):
        pltpu.async_copy(table_hbm.at[idx_v.at[b]], rows_v.at[b], gsem)

    @pl.loop(0, NGROUPS)
    def _(k):
        bank = lax.rem(k, 2) * G
        nbank = G - bank

        for b in range(G):
            pltpu.make_async_copy(
                table_hbm.at[idx_v.at[0]], rows_v.at[0], gsem
            ).wait()

        @pl.when(k >= 1)
        def _():
            for b in range(G):
                pltpu.make_async_copy(
                    rows_v.at[0], out_hbm.at[pl.ds(col, CH)], ssem
                ).wait()

        @pl.when(k + 1 < NGROUPS)
        def _():
            for b in range(G):
                pltpu.async_copy(
                    table_hbm.at[idx_v.at[(k + 1) * G + b]],
                    rows_v.at[nbank + b],
                    gsem,
                )

        for b in range(G):
            pltpu.async_copy(
                rows_v.at[bank + b],
                out_hbm.at[pl.ds((k * G + b) * BATCH + col, CH)],
                ssem,
            )

    for b in range(G):
        pltpu.make_async_copy(
            rows_v.at[0], out_hbm.at[pl.ds(col, CH)], ssem
        ).wait()


def kernel(token_ids, table):
    idx_t = token_ids.T.astype(jnp.int32)
    out = _emb_lookup(idx_t, table)
    return out.reshape(SEQ, BATCH, DIM).transpose(1, 0, 2)


# speedup vs baseline: 2.3433x; 1.1580x over previous
import functools

import jax
import jax.numpy as jnp
from jax import lax
from jax.experimental import pallas as pl
from jax.experimental.pallas import tpu as pltpu
from jax.experimental.pallas import tpu_sc as plsc

BATCH = 4096
SEQ = 50
DIM = 64
PD = 128
B = BATCH * SEQ
NC = 2
NS = 16
NW = NC * NS
CPW = BATCH // NW
CH = CPW
G = 2
NGROUPS = SEQ // G

_mesh = plsc.VectorSubcoreMesh(
    core_axis_name="c", subcore_axis_name="s", num_cores=NC, num_subcores=NS
)


@functools.partial(
    pl.kernel,
    out_type=jax.ShapeDtypeStruct((B, PD), jnp.float32),
    mesh=_mesh,
    scratch_types=[
        pltpu.VMEM((SEQ, CH), jnp.int32),
        pltpu.VMEM((2 * G, CH, PD), jnp.float32),
        pltpu.SemaphoreType.DMA,
        pltpu.SemaphoreType.DMA,
    ],
)
def _emb_lookup(idx_hbm, table_hbm, out_hbm, idx_v, rows_v, gsem, ssem):
    wid = lax.axis_index("s") * NC + lax.axis_index("c")
    col = wid * CPW
    pltpu.sync_copy(idx_hbm.at[:, pl.ds(col, CPW)], idx_v)

    for b in range(G):
        pltpu.async_copy(table_hbm.at[idx_v.at[b]], rows_v.at[b], gsem)

    @pl.loop(0, NGROUPS)
    def _(k):
        bank = lax.rem(k, 2) * G
        nbank = G - bank

        for b in range(G):
            pltpu.make_async_copy(
                table_hbm.at[idx_v.at[0]], rows_v.at[0], gsem
            ).wait()

        @pl.when(k >= 1)
        def _():
            for b in range(G):
                pltpu.make_async_copy(
                    rows_v.at[0], out_hbm.at[pl.ds(col, CH)], ssem
                ).wait()

        @pl.when(k + 1 < NGROUPS)
        def _():
            for b in range(G):
                pltpu.async_copy(
                    table_hbm.at[idx_v.at[(k + 1) * G + b]],
                    rows_v.at[nbank + b],
                    gsem,
                )

        for b in range(G):
            pltpu.async_copy(
                rows_v.at[bank + b],
                out_hbm.at[pl.ds((k * G + b) * BATCH + col, CH)],
                ssem,
            )

    for b in range(G):
        pltpu.make_async_copy(
            rows_v.at[0], out_hbm.at[pl.ds(col, CH)], ssem
        ).wait()


def kernel(token_ids, table):
    idx_t = token_ids.T.astype(jnp.int32)
    tpad = jnp.pad(table, ((0, 0), (0, PD - DIM)))
    out = _emb_lookup(idx_t, tpad)
    return out[:, :DIM].reshape(SEQ, BATCH, DIM).transpose(1, 0, 2)


# speedup vs baseline: 2.3485x; 1.0022x over previous
import functools

import jax
import jax.numpy as jnp
from jax import lax
from jax.experimental import pallas as pl
from jax.experimental.pallas import tpu as pltpu
from jax.experimental.pallas import tpu_sc as plsc

BATCH = 4096
SEQ = 50
DIM = 64
PD = 128
B = BATCH * SEQ
NC = 2
NS = 16
NW = NC * NS
CPW = BATCH // NW
CH = 64
NCH = SEQ * CPW // CH
G = 5
NGROUPS = NCH // G

_mesh = plsc.VectorSubcoreMesh(
    core_axis_name="c", subcore_axis_name="s", num_cores=NC, num_subcores=NS
)


@functools.partial(
    pl.kernel,
    out_type=jax.ShapeDtypeStruct((B, PD), jnp.float32),
    mesh=_mesh,
    scratch_types=[
        pltpu.VMEM((SEQ, CPW), jnp.int32),
        pltpu.VMEM((2 * G, CH, PD), jnp.float32),
        pltpu.SemaphoreType.DMA,
        pltpu.SemaphoreType.DMA,
    ],
)
def _emb_lookup(idx_hbm, table_hbm, out_hbm, idx_v, rows_v, gsem, ssem):
    wid = lax.axis_index("s") * NC + lax.axis_index("c")
    col = wid * CPW
    pltpu.sync_copy(idx_hbm.at[:, pl.ds(col, CPW)], idx_v)

    def idx_slice(c):
        return idx_v.at[c // 2, pl.ds((c % 2) * CH, CH)]

    def out_rows(c):
        return (c // 2) * BATCH + col + (c % 2) * CH

    for b in range(G):
        pltpu.async_copy(table_hbm.at[idx_slice(b)], rows_v.at[b], gsem)

    @pl.loop(0, NGROUPS)
    def _(k):
        bank = lax.rem(k, 2) * G
        nbank = G - bank

        for b in range(G):
            pltpu.make_async_copy(
                table_hbm.at[idx_slice(0)], rows_v.at[0], gsem
            ).wait()

        @pl.when(k >= 1)
        def _():
            for b in range(G):
                pltpu.make_async_copy(
                    rows_v.at[0], out_hbm.at[pl.ds(col, CH)], ssem
                ).wait()

        @pl.when(k + 1 < NGROUPS)
        def _():
            for b in range(G):
                pltpu.async_copy(
                    table_hbm.at[idx_slice((k + 1) * G + b)],
                    rows_v.at[nbank + b],
                    gsem,
                )

        for b in range(G):
            pltpu.async_copy(
                rows_v.at[bank + b],
                out_hbm.at[pl.ds(out_rows(k * G + b), CH)],
                ssem,
            )

    for b in range(G):
        pltpu.make_async_copy(
            rows_v.at[0], out_hbm.at[pl.ds(col, CH)], ssem
        ).wait()


def kernel(token_ids, table):
    idx_t = token_ids.T.astype(jnp.int32)
    tpad = jnp.pad(table, ((0, 0), (0, PD - DIM)))
    out = _emb_lookup(idx_t, tpad)
    return out[:, :DIM].reshape(SEQ, BATCH, DIM).transpose(1, 0, 2)
